# baseline (speedup 1.0000x reference)
import functools

import jax
import jax.numpy as jnp
from jax import lax
from jax.experimental import pallas as pl
from jax.experimental.pallas import tpu as pltpu
from jax.experimental.pallas import tpu_sc as plsc

N = 10000
F = 128
H = 128
R = 8
C = 16
E = 320000

NT = 16
GROUP = 128
EPT = E // NT
CHW = 2048
NFULL = EPT // CHW
REMW = EPT - NFULL * CHW
CAP = GROUP * (-(-EPT // GROUP) + 2)
LHALF = 5120
LROWS = 5248
ROWS_PER_TILE = LROWS // NT
N_CNT = 10240
HCROWS = LHALF // GROUP
BN = 1000


def _phase1(x, conv_weight):
    def body(x_ref, w_ref, o_ref):
        o_ref[0] = jnp.dot(x_ref[...], w_ref[0],
                           preferred_element_type=jnp.float32)

    return pl.pallas_call(
        body,
        grid=(R, N // BN),
        in_specs=[
            pl.BlockSpec((BN, F), lambda r, b: (b, 0)),
            pl.BlockSpec((1, F, H), lambda r, b: (r, 0, 0)),
        ],
        out_specs=pl.BlockSpec((1, BN, H), lambda r, b: (r, b, 0)),
        out_shape=jax.ShapeDtypeStruct((R, N, H), jnp.float32),
    )(x, conv_weight)


def _phase2(h_flat, src, et, dst):
    mesh = plsc.VectorSubcoreMesh(core_axis_name="c", subcore_axis_name="s")

    @functools.partial(
        pl.kernel,
        out_type=(
            jax.ShapeDtypeStruct((2 * LHALF, H), jnp.float32),
            jax.ShapeDtypeStruct((2, HCROWS, GROUP), jnp.float32),
        ),
        mesh=mesh,
        scratch_types=[
            pltpu.VMEM((CHW,), jnp.int32),
            pltpu.VMEM((CHW,), jnp.int32),
            pltpu.VMEM((CHW,), jnp.int32),
            pltpu.VMEM((CAP,), jnp.int32),
            pltpu.VMEM((CAP,), jnp.int32),
            pltpu.VMEM((1, GROUP), jnp.int32),
            pltpu.VMEM((1, GROUP), jnp.int32),
            pltpu.VMEM((GROUP, H), jnp.float32),
            pltpu.VMEM((GROUP, H), jnp.float32),
            pltpu.VMEM((HCROWS, GROUP), jnp.float32),
            pltpu.VMEM((HCROWS,), jnp.int32),
            pltpu.VMEM_SHARED((LROWS, H), jnp.float32),
            pltpu.VMEM_SHARED((HCROWS, GROUP), jnp.float32),
            pltpu.SemaphoreType.DMA,
            pltpu.SemaphoreType.DMA,
        ],
        compiler_params=pltpu.CompilerParams(needs_layout_passes=False),
    )
    def k(h_hbm, src_hbm, et_hbm, dst_hbm, agg_out, cnt_out,
          srcc, etc_, dstc, idxf, dstf, dst2d0, dst2d1, rows0, rows1,
          cnt_v, idxc, agg_sh, cnt_sh, sem, sem_p):
        cid = lax.axis_index("c")
        sid = lax.axis_index("s")
        ebase = sid * EPT

        zero16 = jnp.zeros((16,), jnp.float32)
        zero16i = jnp.zeros((16,), jnp.int32)
        iota16 = lax.iota(jnp.int32, 16)

        def zrow(r, carry):
            for c in range(H // 16):
                rows0[r, pl.ds(c * 16, 16)] = zero16
            return carry

        lax.fori_loop(0, GROUP, zrow, 0)

        def zcnt(r, carry):
            for c in range(GROUP // 16):
                cnt_v[r, pl.ds(c * 16, 16)] = zero16
            return carry

        lax.fori_loop(0, HCROWS, zcnt, 0)
        for t16 in (0, 16, 24):
            idxc[pl.ds(t16, 16)] = iota16 + t16

        def zfill(k16, carry):
            base = k16 * 16
            spread = LHALF + lax.rem(base, GROUP) + iota16
            idxf[pl.ds(base, 16)] = zero16i
            dstf[pl.ds(base, 16)] = spread
            return carry

        lax.fori_loop(0, CAP // 16, zfill, 0)

        rowbase = sid * ROWS_PER_TILE
        nfull = ROWS_PER_TILE // GROUP
        for t in range(nfull):
            pltpu.sync_copy(rows0, agg_sh.at[pl.ds(rowbase + t * GROUP, GROUP)])
        rem = ROWS_PER_TILE - nfull * GROUP
        if rem:
            pltpu.sync_copy(rows0.at[pl.ds(0, rem)],
                            agg_sh.at[pl.ds(rowbase + nfull * GROUP, rem)])

        @pl.when(sid == 0)
        def _():
            pltpu.sync_copy(cnt_v, cnt_sh)

        lo = cid * LHALF

        def vec(v, o):
            s16 = srcc[pl.ds(v * 16, 16)]
            e16 = etc_[pl.ds(v * 16, 16)]
            d16 = dstc[pl.ds(v * 16, 16)]
            local = d16 - lo
            pred = (local >= 0) & (local < LHALF)
            cnts, last = plsc.scan_count(local, mask=pred)
            row = lax.shift_right_logical(local, 7)
            col = lax.bitwise_and(local, GROUP - 1)
            plsc.addupdate_scatter(cnt_v, [row, col],
                                   cnts.astype(jnp.float32), mask=last)
            idx16 = e16 * N + s16
            plsc.store_compressed(idxf.at[pl.ds(o, 16)], idx16, mask=pred)
            plsc.store_compressed(dstf.at[pl.ds(o, 16)], local, mask=pred)
            return o + plsc.all_reduce_population_count(pred)[0]

        def stage(cb, n):
            c1 = pltpu.async_copy(src_hbm.at[pl.ds(cb, n)],
                                  srcc.at[pl.ds(0, n)], sem_p)
            c2 = pltpu.async_copy(et_hbm.at[pl.ds(cb, n)],
                                  etc_.at[pl.ds(0, n)], sem_p)
            c3 = pltpu.async_copy(dst_hbm.at[pl.ds(cb, n)],
                                  dstc.at[pl.ds(0, n)], sem_p)
            c1.wait()
            c2.wait()
            c3.wait()

        def prep(c, o):
            stage(ebase + c * CHW, CHW)
            return lax.fori_loop(0, CHW // 16, vec, o)

        nmine = lax.fori_loop(0, NFULL, prep, jnp.int32(0))
        stage(ebase + NFULL * CHW, REMW)
        nmine = lax.fori_loop(0, REMW // 16, vec, nmine)
        ngroups = lax.div(nmine + (GROUP - 1), jnp.int32(GROUP))
        npairs = lax.div(ngroups + 1, jnp.int32(2))

        pltpu.async_copy(h_hbm.at[idxf.at[pl.ds(0, GROUP)]], rows0, sem)

        plsc.subcore_barrier()

        def fill(d2, g):
            for j in range(GROUP // 16):
                d2[0, pl.ds(j * 16, 16)] = dstf[pl.ds(g * GROUP + j * 16, 16)]

        def pair(h, carry):
            g0 = 2 * h
            g1 = g0 + 1
            fill(dst2d0, g0)
            pltpu.make_async_copy(h_hbm.at[idxf.at[pl.ds(g0 * GROUP, GROUP)]],
                                  rows0, sem).wait()
            pltpu.async_copy(h_hbm.at[idxf.at[pl.ds(g1 * GROUP, GROUP)]],
                             rows1, sem)
            pltpu.sync_copy(rows0, agg_sh.at[dst2d0.at[0]], add=True)
            fill(dst2d1, g1)
            pltpu.make_async_copy(h_hbm.at[idxf.at[pl.ds(g1 * GROUP, GROUP)]],
                                  rows1, sem).wait()
            pltpu.async_copy(h_hbm.at[idxf.at[pl.ds((g0 + 2) * GROUP, GROUP)]],
                             rows0, sem)
            pltpu.sync_copy(rows1, agg_sh.at[dst2d1.at[0]], add=True)
            return carry

        lax.fori_loop(0, npairs, pair, 0)
        pltpu.make_async_copy(h_hbm.at[idxf.at[pl.ds(0, GROUP)]],
                              rows0, sem).wait()

        pltpu.sync_copy(cnt_v, cnt_sh.at[idxc], add=True)
        plsc.subcore_barrier()

        obase = cid * LHALF + rowbase

        @pl.when(sid < NT - 1)
        def _():
            pltpu.sync_copy(agg_sh.at[pl.ds(rowbase, ROWS_PER_TILE)],
                            agg_out.at[pl.ds(obase, ROWS_PER_TILE)])

        LAST = LHALF - (NT - 1) * ROWS_PER_TILE

        @pl.when(sid == NT - 1)
        def _():
            pltpu.sync_copy(agg_sh.at[pl.ds(rowbase, LAST)],
                            agg_out.at[pl.ds(obase, LAST)])

        @pl.when(sid == 0)
        def _():
            pltpu.sync_copy(cnt_sh, cnt_out.at[cid])

    return k(h_flat, src, et, dst)


def _phase3(acc, cnt_col, x, conv_root, conv_bias, lin_W, lin_b):
    def body(a_ref, c_ref, x_ref, root_ref, bias_ref, lw_ref, lb_ref, o_ref):
        cnt = c_ref[...]
        agg = a_ref[...] / jnp.maximum(cnt, 1.0)
        out1 = agg + jnp.dot(x_ref[...], root_ref[...],
                             preferred_element_type=jnp.float32) + bias_ref[...]
        out1 = jnp.maximum(out1, 0.0)
        o_ref[...] = jnp.dot(out1, lw_ref[...],
                             preferred_element_type=jnp.float32) + lb_ref[...]

    return pl.pallas_call(
        body,
        grid=(N // BN,),
        in_specs=[
            pl.BlockSpec((BN, H), lambda b: (b, 0)),
            pl.BlockSpec((BN, 1), lambda b: (b, 0)),
            pl.BlockSpec((BN, F), lambda b: (b, 0)),
            pl.BlockSpec((F, H), lambda b: (0, 0)),
            pl.BlockSpec((1, H), lambda b: (0, 0)),
            pl.BlockSpec((H, C), lambda b: (0, 0)),
            pl.BlockSpec((1, C), lambda b: (0, 0)),
        ],
        out_specs=pl.BlockSpec((BN, C), lambda b: (b, 0)),
        out_shape=jax.ShapeDtypeStruct((N, C), jnp.float32),
    )(acc, cnt_col, x, conv_root, conv_bias, lin_W, lin_b)


def kernel(x, edge_index, edge_type, conv_weight, conv_root, conv_bias, lin_W, lin_b):
    h = _phase1(x, conv_weight)
    h_flat = h.reshape(R * N, H)

    acc, cnt_planes = _phase2(h_flat, edge_index[0], edge_type, edge_index[1])
    cnt_col = cnt_planes.reshape(N_CNT, 1)
    return _phase3(acc, cnt_col, x, conv_root, conv_bias.reshape(1, H),
                   lin_W, lin_b.reshape(1, C))


# speedup vs baseline: 1.6974x; 1.6974x over previous
import functools

import jax
import jax.numpy as jnp
from jax import lax
from jax.experimental import pallas as pl
from jax.experimental.pallas import tpu as pltpu
from jax.experimental.pallas import tpu_sc as plsc

N = 10000
F = 128
H = 128
R = 8
C = 16
E = 320000

NT = 16
GROUP = 128
EPT = E // NT
CHW = 2048
NFULL = EPT // CHW
REMW = EPT - NFULL * CHW
CAP = GROUP * (-(-EPT // GROUP) + 2)
LHALF = 5120
LROWS = 5248
ROWS_PER_TILE = LROWS // NT
N_CNT = 10240
HCROWS = LHALF // GROUP
BN = 1000


def _phase1(x, conv_weight):
    def body(x_ref, w_ref, o_ref):
        xb = x_ref[...]
        for r in range(R):
            o_ref[r] = jnp.dot(xb, w_ref[r],
                               preferred_element_type=jnp.float32)

    return pl.pallas_call(
        body,
        grid=(N // BN,),
        in_specs=[
            pl.BlockSpec((BN, F), lambda b: (b, 0)),
            pl.BlockSpec((R, F, H), lambda b: (0, 0, 0)),
        ],
        out_specs=pl.BlockSpec((R, BN, H), lambda b: (0, b, 0)),
        out_shape=jax.ShapeDtypeStruct((R, N, H), jnp.float32),
    )(x, conv_weight)


def _phase2(h_flat, src, et, dst):
    mesh = plsc.VectorSubcoreMesh(core_axis_name="c", subcore_axis_name="s")

    @functools.partial(
        pl.kernel,
        out_type=(
            jax.ShapeDtypeStruct((2 * LHALF, H), jnp.float32),
            jax.ShapeDtypeStruct((2, HCROWS, GROUP), jnp.float32),
        ),
        mesh=mesh,
        scratch_types=[
            pltpu.VMEM((CHW,), jnp.int32),
            pltpu.VMEM((CHW,), jnp.int32),
            pltpu.VMEM((CHW,), jnp.int32),
            pltpu.VMEM((CAP,), jnp.int32),
            pltpu.VMEM((CAP,), jnp.int32),
            pltpu.VMEM((1, GROUP), jnp.int32),
            pltpu.VMEM((GROUP, H), jnp.float32),
            pltpu.VMEM((HCROWS, GROUP), jnp.float32),
            pltpu.VMEM((HCROWS,), jnp.int32),
            pltpu.VMEM_SHARED((LROWS, H), jnp.float32),
            pltpu.VMEM_SHARED((HCROWS, GROUP), jnp.float32),
            pltpu.SemaphoreType.DMA,
            pltpu.SemaphoreType.DMA,
        ],
        compiler_params=pltpu.CompilerParams(needs_layout_passes=False),
    )
    def k(h_hbm, src_hbm, et_hbm, dst_hbm, agg_out, cnt_out,
          srcc, etc_, dstc, idxf, dstf, dst2d0, rows0,
          cnt_v, idxc, agg_sh, cnt_sh, sem, sem_p):
        cid = lax.axis_index("c")
        sid = lax.axis_index("s")
        ebase = sid * EPT

        zero16 = jnp.zeros((16,), jnp.float32)
        zero16i = jnp.zeros((16,), jnp.int32)
        iota16 = lax.iota(jnp.int32, 16)

        def zrow(r, carry):
            for c in range(H // 16):
                rows0[r, pl.ds(c * 16, 16)] = zero16
            return carry

        lax.fori_loop(0, GROUP, zrow, 0)

        def zcnt(r, carry):
            for c in range(GROUP // 16):
                cnt_v[r, pl.ds(c * 16, 16)] = zero16
            return carry

        lax.fori_loop(0, HCROWS, zcnt, 0)
        for t16 in (0, 16, 24):
            idxc[pl.ds(t16, 16)] = iota16 + t16

        def zfill(k16, carry):
            base = k16 * 16
            spread = LHALF + lax.rem(base, GROUP) + iota16
            idxf[pl.ds(base, 16)] = zero16i
            dstf[pl.ds(base, 16)] = spread
            return carry

        lax.fori_loop(0, CAP // 16, zfill, 0)

        rowbase = sid * ROWS_PER_TILE
        nfull = ROWS_PER_TILE // GROUP
        for t in range(nfull):
            pltpu.sync_copy(rows0, agg_sh.at[pl.ds(rowbase + t * GROUP, GROUP)])
        rem = ROWS_PER_TILE - nfull * GROUP
        if rem:
            pltpu.sync_copy(rows0.at[pl.ds(0, rem)],
                            agg_sh.at[pl.ds(rowbase + nfull * GROUP, rem)])

        @pl.when(sid == 0)
        def _():
            pltpu.sync_copy(cnt_v, cnt_sh)

        lo = cid * LHALF

        def vec(v, o):
            s16 = srcc[pl.ds(v * 16, 16)]
            e16 = etc_[pl.ds(v * 16, 16)]
            d16 = dstc[pl.ds(v * 16, 16)]
            local = d16 - lo
            pred = (local >= 0) & (local < LHALF)
            cnts, last = plsc.scan_count(local, mask=pred)
            row = lax.shift_right_logical(local, 7)
            col = lax.bitwise_and(local, GROUP - 1)
            plsc.addupdate_scatter(cnt_v, [row, col],
                                   cnts.astype(jnp.float32), mask=last)
            idx16 = e16 * N + s16
            plsc.store_compressed(idxf.at[pl.ds(o, 16)], idx16, mask=pred)
            plsc.store_compressed(dstf.at[pl.ds(o, 16)], local, mask=pred)
            return o + plsc.all_reduce_population_count(pred)[0]

        def stage(cb, n):
            c1 = pltpu.async_copy(src_hbm.at[pl.ds(cb, n)],
                                  srcc.at[pl.ds(0, n)], sem_p)
            c2 = pltpu.async_copy(et_hbm.at[pl.ds(cb, n)],
                                  etc_.at[pl.ds(0, n)], sem_p)
            c3 = pltpu.async_copy(dst_hbm.at[pl.ds(cb, n)],
                                  dstc.at[pl.ds(0, n)], sem_p)
            c1.wait()
            c2.wait()
            c3.wait()

        def prep(c, o):
            stage(ebase + c * CHW, CHW)
            return lax.fori_loop(0, CHW // 16, vec, o)

        nmine = lax.fori_loop(0, NFULL, prep, jnp.int32(0))
        stage(ebase + NFULL * CHW, REMW)
        nmine = lax.fori_loop(0, REMW // 16, vec, nmine)
        ngroups = lax.div(nmine + (GROUP - 1), jnp.int32(GROUP))

        plsc.subcore_barrier()

        def gbody(g, carry):
            cp = pltpu.async_copy(h_hbm.at[idxf.at[pl.ds(g * GROUP, GROUP)]],
                                  rows0, sem)
            for j in range(GROUP // 16):
                dst2d0[0, pl.ds(j * 16, 16)] = dstf[pl.ds(g * GROUP + j * 16, 16)]
            cp.wait()
            pltpu.sync_copy(rows0, agg_sh.at[dst2d0.at[0]], add=True)
            return carry

        lax.fori_loop(0, ngroups, gbody, 0)

        pltpu.sync_copy(cnt_v, cnt_sh.at[idxc], add=True)
        plsc.subcore_barrier()

        obase = cid * LHALF + rowbase

        @pl.when(sid < NT - 1)
        def _():
            pltpu.sync_copy(agg_sh.at[pl.ds(rowbase, ROWS_PER_TILE)],
                            agg_out.at[pl.ds(obase, ROWS_PER_TILE)])

        LAST = LHALF - (NT - 1) * ROWS_PER_TILE

        @pl.when(sid == NT - 1)
        def _():
            pltpu.sync_copy(agg_sh.at[pl.ds(rowbase, LAST)],
                            agg_out.at[pl.ds(obase, LAST)])

        @pl.when(sid == 0)
        def _():
            pltpu.sync_copy(cnt_sh, cnt_out.at[cid])

    return k(h_flat, src, et, dst)


def _phase3(acc, cnt_col, x, conv_root, conv_bias, lin_W, lin_b):
    def body(a_ref, c_ref, x_ref, root_ref, bias_ref, lw_ref, lb_ref, o_ref):
        cnt = c_ref[...]
        agg = a_ref[...] / jnp.maximum(cnt, 1.0)
        out1 = agg + jnp.dot(x_ref[...], root_ref[...],
                             preferred_element_type=jnp.float32) + bias_ref[...]
        out1 = jnp.maximum(out1, 0.0)
        o_ref[...] = jnp.dot(out1, lw_ref[...],
                             preferred_element_type=jnp.float32) + lb_ref[...]

    return pl.pallas_call(
        body,
        grid=(N // BN,),
        in_specs=[
            pl.BlockSpec((BN, H), lambda b: (b, 0)),
            pl.BlockSpec((BN, 1), lambda b: (b, 0)),
            pl.BlockSpec((BN, F), lambda b: (b, 0)),
            pl.BlockSpec((F, H), lambda b: (0, 0)),
            pl.BlockSpec((1, H), lambda b: (0, 0)),
            pl.BlockSpec((H, C), lambda b: (0, 0)),
            pl.BlockSpec((1, C), lambda b: (0, 0)),
        ],
        out_specs=pl.BlockSpec((BN, C), lambda b: (b, 0)),
        out_shape=jax.ShapeDtypeStruct((N, C), jnp.float32),
    )(acc, cnt_col, x, conv_root, conv_bias, lin_W, lin_b)


def kernel(x, edge_index, edge_type, conv_weight, conv_root, conv_bias, lin_W, lin_b):
    h = _phase1(x, conv_weight)
    h_flat = h.reshape(R * N, H)

    acc, cnt_planes = _phase2(h_flat, edge_index[0], edge_type, edge_index[1])
    cnt_col = cnt_planes.reshape(N_CNT, 1)
    return _phase3(acc, cnt_col, x, conv_root, conv_bias.reshape(1, H),
                   lin_W, lin_b.reshape(1, C))


# speedup vs baseline: 1.7330x; 1.0210x over previous
import functools

import jax
import jax.numpy as jnp
from jax import lax
from jax.experimental import pallas as pl
from jax.experimental.pallas import tpu as pltpu
from jax.experimental.pallas import tpu_sc as plsc

N = 10000
F = 128
H = 128
R = 8
C = 16
E = 320000

NT = 16
GROUP = 128
EPT = E // NT
CHW = 2048
NFULL = EPT // CHW
REMW = EPT - NFULL * CHW
CAP = GROUP * (-(-EPT // GROUP) + 2)
LHALF = 5120
LROWS = 5248
ROWS_PER_TILE = LROWS // NT
N_CNT = 10240
HCROWS = LHALF // GROUP
BN = 1000


def _phase1(x, conv_weight):
    def body(x_ref, w_ref, o_ref):
        xb = x_ref[...]
        for r in range(R):
            o_ref[r] = jnp.dot(xb, w_ref[r],
                               preferred_element_type=jnp.float32)

    return pl.pallas_call(
        body,
        grid=(N // BN,),
        in_specs=[
            pl.BlockSpec((BN, F), lambda b: (b, 0)),
            pl.BlockSpec((R, F, H), lambda b: (0, 0, 0)),
        ],
        out_specs=pl.BlockSpec((R, BN, H), lambda b: (0, b, 0)),
        out_shape=jax.ShapeDtypeStruct((R, N, H), jnp.float32),
    )(x, conv_weight)


def _phase2(h_flat, src, et, dst):
    mesh = plsc.VectorSubcoreMesh(core_axis_name="c", subcore_axis_name="s")

    @functools.partial(
        pl.kernel,
        out_type=(
            jax.ShapeDtypeStruct((2 * LHALF, H), jnp.float32),
            jax.ShapeDtypeStruct((2, HCROWS, GROUP), jnp.float32),
        ),
        mesh=mesh,
        scratch_types=[
            pltpu.VMEM((CHW,), jnp.int32),
            pltpu.VMEM((CHW,), jnp.int32),
            pltpu.VMEM((CHW,), jnp.int32),
            pltpu.VMEM((CAP,), jnp.int32),
            pltpu.VMEM((CAP,), jnp.int32),
            pltpu.VMEM((1, GROUP), jnp.int32),
            pltpu.VMEM((GROUP, H), jnp.float32),
            pltpu.VMEM((HCROWS, GROUP), jnp.float32),
            pltpu.VMEM((HCROWS,), jnp.int32),
            pltpu.VMEM_SHARED((LROWS, H), jnp.float32),
            pltpu.VMEM_SHARED((HCROWS, GROUP), jnp.float32),
            pltpu.SemaphoreType.DMA,
            pltpu.SemaphoreType.DMA,
        ],
        compiler_params=pltpu.CompilerParams(needs_layout_passes=False),
    )
    def k(h_hbm, src_hbm, et_hbm, dst_hbm, agg_out, cnt_out,
          srcc, etc_, dstc, idxf, dstf, dst2d0, rows0,
          cnt_v, idxc, agg_sh, cnt_sh, sem, sem_p):
        cid = lax.axis_index("c")
        sid = lax.axis_index("s")
        ebase = sid * EPT

        zero16 = jnp.zeros((16,), jnp.float32)
        zero16i = jnp.zeros((16,), jnp.int32)
        iota16 = lax.iota(jnp.int32, 16)

        def zrow(r, carry):
            for c in range(H // 16):
                rows0[r, pl.ds(c * 16, 16)] = zero16
            return carry

        lax.fori_loop(0, GROUP, zrow, 0)

        def zcnt(r, carry):
            for c in range(GROUP // 16):
                cnt_v[r, pl.ds(c * 16, 16)] = zero16
            return carry

        lax.fori_loop(0, HCROWS, zcnt, 0)
        for t16 in (0, 16, 24):
            idxc[pl.ds(t16, 16)] = iota16 + t16

        def zfill(k16, carry):
            base = k16 * 16
            spread = LHALF + lax.rem(base, GROUP) + iota16
            idxf[pl.ds(base, 16)] = zero16i
            dstf[pl.ds(base, 16)] = spread
            return carry

        lax.fori_loop(0, CAP // 16, zfill, 0)

        rowbase = sid * ROWS_PER_TILE
        nfull = ROWS_PER_TILE // GROUP
        for t in range(nfull):
            pltpu.sync_copy(rows0, agg_sh.at[pl.ds(rowbase + t * GROUP, GROUP)])
        rem = ROWS_PER_TILE - nfull * GROUP
        if rem:
            pltpu.sync_copy(rows0.at[pl.ds(0, rem)],
                            agg_sh.at[pl.ds(rowbase + nfull * GROUP, rem)])

        @pl.when(sid == 0)
        def _():
            pltpu.sync_copy(cnt_v, cnt_sh)

        lo = cid * LHALF

        def vec(v, o):
            s16 = srcc[pl.ds(v * 16, 16)]
            e16 = etc_[pl.ds(v * 16, 16)]
            d16 = dstc[pl.ds(v * 16, 16)]
            local = d16 - lo
            pred = (local >= 0) & (local < LHALF)
            idx16 = e16 * N + s16
            plsc.store_compressed(idxf.at[pl.ds(o, 16)], idx16, mask=pred)
            plsc.store_compressed(dstf.at[pl.ds(o, 16)], local, mask=pred)
            return o + plsc.all_reduce_population_count(pred)[0]

        def stage(cb, n):
            c1 = pltpu.async_copy(src_hbm.at[pl.ds(cb, n)],
                                  srcc.at[pl.ds(0, n)], sem_p)
            c2 = pltpu.async_copy(et_hbm.at[pl.ds(cb, n)],
                                  etc_.at[pl.ds(0, n)], sem_p)
            c3 = pltpu.async_copy(dst_hbm.at[pl.ds(cb, n)],
                                  dstc.at[pl.ds(0, n)], sem_p)
            c1.wait()
            c2.wait()
            c3.wait()

        def prep(c, o):
            stage(ebase + c * CHW, CHW)
            return lax.fori_loop(0, CHW // 16, vec, o)

        nmine = lax.fori_loop(0, NFULL, prep, jnp.int32(0))
        stage(ebase + NFULL * CHW, REMW)
        nmine = lax.fori_loop(0, REMW // 16, vec, nmine)
        ngroups = lax.div(nmine + (GROUP - 1), jnp.int32(GROUP))

        plsc.subcore_barrier()

        def gbody(g, carry):
            cp = pltpu.async_copy(h_hbm.at[idxf.at[pl.ds(g * GROUP, GROUP)]],
                                  rows0, sem)
            for j in range(GROUP // 16):
                v16 = dstf[pl.ds(g * GROUP + j * 16, 16)]
                dst2d0[0, pl.ds(j * 16, 16)] = v16
                pred = v16 < LHALF
                cnts, last = plsc.scan_count(v16, mask=pred)
                row = lax.shift_right_logical(v16, 7)
                col = lax.bitwise_and(v16, GROUP - 1)
                plsc.addupdate_scatter(cnt_v, [row, col],
                                       cnts.astype(jnp.float32), mask=last)
            cp.wait()
            pltpu.sync_copy(rows0, agg_sh.at[dst2d0.at[0]], add=True)
            return carry

        lax.fori_loop(0, ngroups, gbody, 0)

        pltpu.sync_copy(cnt_v, cnt_sh.at[idxc], add=True)
        plsc.subcore_barrier()

        obase = cid * LHALF + rowbase

        @pl.when(sid < NT - 1)
        def _():
            pltpu.sync_copy(agg_sh.at[pl.ds(rowbase, ROWS_PER_TILE)],
                            agg_out.at[pl.ds(obase, ROWS_PER_TILE)])

        LAST = LHALF - (NT - 1) * ROWS_PER_TILE

        @pl.when(sid == NT - 1)
        def _():
            pltpu.sync_copy(agg_sh.at[pl.ds(rowbase, LAST)],
                            agg_out.at[pl.ds(obase, LAST)])

        @pl.when(sid == 0)
        def _():
            pltpu.sync_copy(cnt_sh, cnt_out.at[cid])

    return k(h_flat, src, et, dst)


def _phase3(acc, cnt_col, x, conv_root, conv_bias, lin_W, lin_b):
    def body(a_ref, c_ref, x_ref, root_ref, bias_ref, lw_ref, lb_ref, o_ref):
        cnt = c_ref[...]
        agg = a_ref[...] / jnp.maximum(cnt, 1.0)
        out1 = agg + jnp.dot(x_ref[...], root_ref[...],
                             preferred_element_type=jnp.float32) + bias_ref[...]
        out1 = jnp.maximum(out1, 0.0)
        o_ref[...] = jnp.dot(out1, lw_ref[...],
                             preferred_element_type=jnp.float32) + lb_ref[...]

    return pl.pallas_call(
        body,
        grid=(N // BN,),
        in_specs=[
            pl.BlockSpec((BN, H), lambda b: (b, 0)),
            pl.BlockSpec((BN, 1), lambda b: (b, 0)),
            pl.BlockSpec((BN, F), lambda b: (b, 0)),
            pl.BlockSpec((F, H), lambda b: (0, 0)),
            pl.BlockSpec((1, H), lambda b: (0, 0)),
            pl.BlockSpec((H, C), lambda b: (0, 0)),
            pl.BlockSpec((1, C), lambda b: (0, 0)),
        ],
        out_specs=pl.BlockSpec((BN, C), lambda b: (b, 0)),
        out_shape=jax.ShapeDtypeStruct((N, C), jnp.float32),
    )(acc, cnt_col, x, conv_root, conv_bias, lin_W, lin_b)


def kernel(x, edge_index, edge_type, conv_weight, conv_root, conv_bias, lin_W, lin_b):
    h = _phase1(x, conv_weight)
    h_flat = h.reshape(R * N, H)

    acc, cnt_planes = _phase2(h_flat, edge_index[0], edge_type, edge_index[1])
    cnt_col = cnt_planes.reshape(N_CNT, 1)
    return _phase3(acc, cnt_col, x, conv_root, conv_bias.reshape(1, H),
                   lin_W, lin_b.reshape(1, C))
